# baseline (speedup 1.0000x reference)
import functools

import jax
import jax.numpy as jnp
from jax import lax
from jax.experimental import pallas as pl
from jax.experimental.pallas import tpu as pltpu
from jax.experimental.pallas import tpu_sc as plsc

B, C, N, K = 16, 3, 2048, 20
T = 256
NT = N // T
T1 = 512
NT1 = N // T1
HALF = (N // 2) * K
_NEG = float("-inf")


def _tsum(x):
    return jnp.sum(jnp.sum(x, axis=1, keepdims=True), axis=0, keepdims=True)


def _stats_row(svals):
    return jnp.concatenate(svals, axis=1)


def _knn_body(x_ref, xtt_ref, idx_ref):
    xall = x_ref[0]
    xrow_t = xtt_ref[0]
    xxall = jnp.sum(xall * xall, axis=0, keepdims=True)
    xxrow = jnp.sum(xrow_t * xrow_t, axis=1, keepdims=True)
    inner = jax.lax.dot_general(
        xrow_t, xall, (((1,), (0,)), ((), ())),
        preferred_element_type=jnp.float32)
    inner = -2.0 * inner
    vals = (-xxall - inner) - xxrow

    lane_iota = jax.lax.broadcasted_iota(jnp.int32, (T1, N), 1)
    k_iota = jax.lax.broadcasted_iota(jnp.int32, (T1, K), 1)
    iacc = jnp.zeros((T1, K), jnp.int32)
    for j in range(K):
        a = jnp.argmax(vals, axis=1, keepdims=True).astype(jnp.int32)
        iacc = iacc + jnp.where(k_iota == j, a, 0)
        vals = jnp.where(lane_iota == a, _NEG, vals)
    idx_ref[0] = iacc


def _gather_tec(x_hbm, idx_hbm, nb_hbm, xp0, xp1, xp2, idx_v, ob0, ob1, ob2):
    wid = lax.axis_index("s") * 2 + lax.axis_index("c")
    b = wid // 2
    base = (wid % 2) * HALF
    pltpu.sync_copy(x_hbm.at[pl.ds((b * C + 0) * N, N)], xp0)
    pltpu.sync_copy(x_hbm.at[pl.ds((b * C + 1) * N, N)], xp1)
    pltpu.sync_copy(x_hbm.at[pl.ds((b * C + 2) * N, N)], xp2)
    pltpu.sync_copy(idx_hbm.at[pl.ds(b * (N * K) + base, HALF)], idx_v)

    def body(i, carry):
        sl = pl.ds(i * 16, 16)
        iv = idx_v[sl]
        ob0[sl] = plsc.load_gather(xp0, [iv])
        ob1[sl] = plsc.load_gather(xp1, [iv])
        ob2[sl] = plsc.load_gather(xp2, [iv])
        return carry

    lax.fori_loop(0, HALF // 16, body, 0)
    pltpu.sync_copy(ob0, nb_hbm.at[pl.ds((b * C + 0) * (N * K) + base, HALF)])
    pltpu.sync_copy(ob1, nb_hbm.at[pl.ds((b * C + 1) * (N * K) + base, HALF)])
    pltpu.sync_copy(ob2, nb_hbm.at[pl.ds((b * C + 2) * (N * K) + base, HALF)])


def _sc_gather(x, idxf):
    f = pl.kernel(
        _gather_tec,
        out_type=jax.ShapeDtypeStruct((B * C * N * K,), jnp.float32),
        mesh=plsc.VectorSubcoreMesh(core_axis_name="c", subcore_axis_name="s"),
        compiler_params=pltpu.CompilerParams(needs_layout_passes=False),
        scratch_types=[
            pltpu.VMEM((N,), jnp.float32),
            pltpu.VMEM((N,), jnp.float32),
            pltpu.VMEM((N,), jnp.float32),
            pltpu.VMEM((HALF,), jnp.int32),
            pltpu.VMEM((HALF,), jnp.float32),
            pltpu.VMEM((HALF,), jnp.float32),
            pltpu.VMEM((HALF,), jnp.float32),
        ],
    )
    return f(x.reshape(B * C * N), idxf.reshape(B * N * K))


def _xd_tile(nb_ref, xrow_t):
    return [xrow_t[:, c:c + 1] - nb_ref[0, c] for c in range(3)]


def _mom_body(nb_ref, xtt_ref, st_ref):
    xds = _xd_tile(nb_ref, xtt_ref[0])
    st_ref[0, 0, 0:1, 0:9] = _stats_row(
        [_tsum(xds[0]), _tsum(xds[1]), _tsum(xds[2]),
         _tsum(xds[0] * xds[0]), _tsum(xds[0] * xds[1]),
         _tsum(xds[0] * xds[2]), _tsum(xds[1] * xds[1]),
         _tsum(xds[1] * xds[2]), _tsum(xds[2] * xds[2])])


def _feat_body(nb_ref, xtt_ref, pk_ref, ef_ref, ya_ref, yb_ref, st_ref):
    pk = pk_ref[...]
    xd = _xd_tile(nb_ref, xtt_ref[0])
    ya = jnp.zeros((T, K), jnp.float32)
    yb = jnp.zeros((T, K), jnp.float32)
    for o in range(16):
        nf = (pk[3:4, o:o + 1]
              + pk[0:1, o:o + 1] * xd[0]
              + pk[1:2, o:o + 1] * xd[1]
              + pk[2:3, o:o + 1] * xd[2])
        ef = (pk[7:8, o:o + 1]
              + pk[4:5, o:o + 1] * xd[0]
              + pk[5:6, o:o + 1] * xd[1]
              + pk[6:7, o:o + 1] * xd[2])
        nf = jnp.maximum(nf, 0.0)
        ef = jnp.maximum(ef, 0.0)
        ef_ref[0, o] = ef
        w3o = pk[8:9, o:o + 1]
        ya = ya + w3o * nf
        yb = yb + w3o * ef
    b3s = pk[9:10, 0:1]
    ya = ya + b3s
    yb = yb + b3s
    ya_ref[0] = ya
    yb_ref[0] = yb
    st_ref[0, 0, 0:1, 0:4] = _stats_row(
        [_tsum(ya), _tsum(ya * ya), _tsum(yb), _tsum(yb * yb)])


def _attn_body(nb_ref, xtt_ref, ya_ref, yb_ref, pk_ref, out_ref):
    pk = pk_ref[...]
    ya = ya_ref[0]
    yb = yb_ref[0]
    sa = jnp.maximum(pk[10:11, 0:1] * ya + pk[10:11, 1:2], 0.0)
    na = jnp.maximum(pk[10:11, 2:3] * yb + pk[10:11, 3:4], 0.0)
    lg = sa + na
    lr = jnp.where(lg >= 0, lg, 0.01 * lg)
    mx = jnp.max(lr, axis=1, keepdims=True)
    e = jnp.exp(lr - mx)
    pr = e / jnp.sum(e, axis=1, keepdims=True)
    xd = _xd_tile(nb_ref, xtt_ref[0])
    cols = []
    for o in range(16):
        ef = (pk[7:8, o:o + 1]
              + pk[4:5, o:o + 1] * xd[0]
              + pk[5:6, o:o + 1] * xd[1]
              + pk[6:7, o:o + 1] * xd[2])
        ef = jnp.maximum(ef, 0.0)
        cols.append(jnp.sum(pr * ef, axis=1, keepdims=True))
    v = jnp.concatenate(cols, axis=1)
    out_ref[0] = jnp.where(v > 0, v, jnp.exp(v) - 1.0)


def kernel(x, n_neighbor, W1, g1, be1, W2, b2, g2, be2, W3, b3, g3, be3):
    x = x.astype(jnp.float32)
    xt = jnp.transpose(x, (0, 2, 1))

    idx = pl.pallas_call(
        _knn_body,
        grid=(B, NT1),
        in_specs=[
            pl.BlockSpec((1, C, N), lambda b, t: (b, 0, 0)),
            pl.BlockSpec((1, T1, C), lambda b, t: (b, t, 0)),
        ],
        out_specs=pl.BlockSpec((1, T1, K), lambda b, t: (b, t, 0)),
        out_shape=jax.ShapeDtypeStruct((B, N, K), jnp.int32),
    )(x, xt)

    nb = _sc_gather(x, idx).reshape(B, C, N, K)

    nb_spec = pl.BlockSpec((1, C, T, K), lambda b, t: (b, 0, t, 0))
    xtt_spec = pl.BlockSpec((1, T, C), lambda b, t: (b, t, 0))
    st_spec = pl.BlockSpec((1, 1, 8, 128), lambda b, t: (b, t, 0, 0))
    st_shape = jax.ShapeDtypeStruct((B, NT, 8, 128), jnp.float32)
    pk_spec = pl.BlockSpec((16, 16), lambda b, t: (0, 0))

    st1 = pl.pallas_call(
        _mom_body,
        grid=(B, NT),
        in_specs=[nb_spec, xtt_spec],
        out_specs=st_spec,
        out_shape=st_shape,
    )(nb, xt)

    cnt = jnp.float32(B * N * K)
    s = jnp.sum(st1[:, :, 0, :9], axis=(0, 1))
    mu = s[:3] / cnt
    q = s[3:9] / cnt
    S = jnp.stack([
        jnp.stack([q[0], q[1], q[2]]),
        jnp.stack([q[1], q[3], q[4]]),
        jnp.stack([q[2], q[4], q[5]]),
    ])
    mean1 = W1 @ mu
    var1 = jnp.sum((W1 @ S) * W1, axis=1) - mean1 ** 2
    a1 = g1 / jnp.sqrt(var1 + 1e-5)
    A1 = a1[:, None] * W1
    beta1 = be1 - mean1 * a1
    z2 = W2 @ mu
    mean2 = z2 + b2
    var2 = jnp.sum((W2 @ S) * W2, axis=1) - z2 ** 2
    a2 = g2 / jnp.sqrt(var2 + 1e-5)
    A2 = a2[:, None] * W2
    beta2 = a2 * b2 + be2 - mean2 * a2

    pk = jnp.zeros((16, 16), jnp.float32)
    pk = pk.at[0:3, :].set(A1.T)
    pk = pk.at[3, :].set(beta1)
    pk = pk.at[4:7, :].set(A2.T)
    pk = pk.at[7, :].set(beta2)
    pk = pk.at[8, :].set(W3[0])
    pk = pk.at[9, 0].set(b3[0])

    ef, ya, yb, st2 = pl.pallas_call(
        _feat_body,
        grid=(B, NT),
        in_specs=[nb_spec, xtt_spec, pk_spec],
        out_specs=[
            pl.BlockSpec((1, 16, T, K), lambda b, t: (b, 0, t, 0)),
            pl.BlockSpec((1, T, K), lambda b, t: (b, t, 0)),
            pl.BlockSpec((1, T, K), lambda b, t: (b, t, 0)),
            st_spec,
        ],
        out_shape=[
            jax.ShapeDtypeStruct((B, 16, N, K), jnp.float32),
            jax.ShapeDtypeStruct((B, N, K), jnp.float32),
            jax.ShapeDtypeStruct((B, N, K), jnp.float32),
            st_shape,
        ],
    )(nb, xt, pk)

    s2 = jnp.sum(st2[:, :, 0, :4], axis=(0, 1))
    m3a = s2[0] / cnt
    v3a = s2[1] / cnt - m3a ** 2
    a3a = g3[0] / jnp.sqrt(v3a + 1e-5)
    b3a = be3[0] - m3a * a3a
    m3b = s2[2] / cnt
    v3b = s2[3] / cnt - m3b ** 2
    a3b = g3[0] / jnp.sqrt(v3b + 1e-5)
    b3b = be3[0] - m3b * a3b
    pk = pk.at[10, 0:4].set(jnp.stack([a3a, b3a, a3b, b3b]))

    out = pl.pallas_call(
        _attn_body,
        grid=(B, NT),
        in_specs=[
            nb_spec,
            xtt_spec,
            pl.BlockSpec((1, T, K), lambda b, t: (b, t, 0)),
            pl.BlockSpec((1, T, K), lambda b, t: (b, t, 0)),
            pk_spec,
        ],
        out_specs=pl.BlockSpec((1, T, 16), lambda b, t: (b, t, 0)),
        out_shape=jax.ShapeDtypeStruct((B, N, 16), jnp.float32),
    )(nb, xt, ya, yb, pk)

    ret = out[:, :, None, :]
    return (ret, ef)


# speedup vs baseline: 1.0837x; 1.0837x over previous
import functools

import jax
import jax.numpy as jnp
from jax import lax
from jax.experimental import pallas as pl
from jax.experimental.pallas import tpu as pltpu
from jax.experimental.pallas import tpu_sc as plsc

B, C, N, K = 16, 3, 2048, 20
T = 256
NT = N // T
T1 = 512
NT1 = N // T1
HALF = (N // 2) * K
_NEG = float("-inf")


def _tsum(x):
    return jnp.sum(jnp.sum(x, axis=1, keepdims=True), axis=0, keepdims=True)


def _stats_row(svals):
    return jnp.concatenate(svals, axis=1)


def _knn_body(x_ref, xtt_ref, idx_ref):
    xall = x_ref[0]
    xrow_t = xtt_ref[0]
    xxall = jnp.sum(xall * xall, axis=0, keepdims=True)
    xxrow = jnp.sum(xrow_t * xrow_t, axis=1, keepdims=True)
    inner = jax.lax.dot_general(
        xrow_t, xall, (((1,), (0,)), ((), ())),
        preferred_element_type=jnp.float32)
    inner = -2.0 * inner
    vals = (-xxall - inner) - xxrow

    lane_iota = jax.lax.broadcasted_iota(jnp.int32, (T1, N), 1)
    k_iota = jax.lax.broadcasted_iota(jnp.int32, (T1, K), 1)
    iacc = jnp.zeros((T1, K), jnp.int32)
    for j in range(K):
        a = jnp.argmax(vals, axis=1, keepdims=True).astype(jnp.int32)
        iacc = iacc + jnp.where(k_iota == j, a, 0)
        vals = jnp.where(lane_iota == a, _NEG, vals)
    idx_ref[0] = iacc


def _gather_tec(x_hbm, idx_hbm, cidx_hbm, xd_hbm, mom_hbm,
                xp0, xp1, xp2, idx_v, cid_v, ob0, ob1, ob2, mom_v):
    wid = lax.axis_index("s") * 2 + lax.axis_index("c")
    b = wid // 2
    base = (wid % 2) * HALF
    pltpu.sync_copy(x_hbm.at[pl.ds((b * C + 0) * N, N)], xp0)
    pltpu.sync_copy(x_hbm.at[pl.ds((b * C + 1) * N, N)], xp1)
    pltpu.sync_copy(x_hbm.at[pl.ds((b * C + 2) * N, N)], xp2)
    pltpu.sync_copy(idx_hbm.at[pl.ds(b * (N * K) + base, HALF)], idx_v)
    pltpu.sync_copy(cidx_hbm.at[pl.ds(base, HALF)], cid_v)

    def body(i, acc):
        sl = pl.ds(i * 16, 16)
        iv = idx_v[sl]
        civ = cid_v[sl]
        xd0 = plsc.load_gather(xp0, [civ]) - plsc.load_gather(xp0, [iv])
        xd1 = plsc.load_gather(xp1, [civ]) - plsc.load_gather(xp1, [iv])
        xd2 = plsc.load_gather(xp2, [civ]) - plsc.load_gather(xp2, [iv])
        ob0[sl] = xd0
        ob1[sl] = xd1
        ob2[sl] = xd2
        return (acc[0] + xd0, acc[1] + xd1, acc[2] + xd2,
                acc[3] + xd0 * xd0, acc[4] + xd0 * xd1, acc[5] + xd0 * xd2,
                acc[6] + xd1 * xd1, acc[7] + xd1 * xd2, acc[8] + xd2 * xd2)

    zero = jnp.zeros((16,), jnp.float32)
    acc = lax.fori_loop(0, HALF // 16, body, (zero,) * 9)
    for i in range(9):
        mom_v[pl.ds(i * 16, 16)] = acc[i]
    pltpu.sync_copy(ob0, xd_hbm.at[pl.ds((b * C + 0) * (N * K) + base, HALF)])
    pltpu.sync_copy(ob1, xd_hbm.at[pl.ds((b * C + 1) * (N * K) + base, HALF)])
    pltpu.sync_copy(ob2, xd_hbm.at[pl.ds((b * C + 2) * (N * K) + base, HALF)])
    pltpu.sync_copy(mom_v, mom_hbm.at[pl.ds(wid * 144, 144)])


def _sc_gather(x, idxf, cidx):
    f = pl.kernel(
        _gather_tec,
        out_type=[
            jax.ShapeDtypeStruct((B * C * N * K,), jnp.float32),
            jax.ShapeDtypeStruct((32 * 144,), jnp.float32),
        ],
        mesh=plsc.VectorSubcoreMesh(core_axis_name="c", subcore_axis_name="s"),
        compiler_params=pltpu.CompilerParams(needs_layout_passes=False),
        scratch_types=[
            pltpu.VMEM((N,), jnp.float32),
            pltpu.VMEM((N,), jnp.float32),
            pltpu.VMEM((N,), jnp.float32),
            pltpu.VMEM((HALF,), jnp.int32),
            pltpu.VMEM((HALF,), jnp.int32),
            pltpu.VMEM((HALF,), jnp.float32),
            pltpu.VMEM((HALF,), jnp.float32),
            pltpu.VMEM((HALF,), jnp.float32),
            pltpu.VMEM((144,), jnp.float32),
        ],
    )
    return f(x.reshape(B * C * N), idxf.reshape(B * N * K), cidx)


def _feat_body(xd_ref, pk_ref, ef_ref, ya_ref, yb_ref, st_ref):
    pk = pk_ref[...]
    xd = [xd_ref[0, 0], xd_ref[0, 1], xd_ref[0, 2]]
    ya = jnp.zeros((T, K), jnp.float32)
    yb = jnp.zeros((T, K), jnp.float32)
    for o in range(16):
        nf = (pk[3:4, o:o + 1]
              + pk[0:1, o:o + 1] * xd[0]
              + pk[1:2, o:o + 1] * xd[1]
              + pk[2:3, o:o + 1] * xd[2])
        ef = (pk[7:8, o:o + 1]
              + pk[4:5, o:o + 1] * xd[0]
              + pk[5:6, o:o + 1] * xd[1]
              + pk[6:7, o:o + 1] * xd[2])
        nf = jnp.maximum(nf, 0.0)
        ef = jnp.maximum(ef, 0.0)
        ef_ref[0, o] = ef
        w3o = pk[8:9, o:o + 1]
        ya = ya + w3o * nf
        yb = yb + w3o * ef
    b3s = pk[9:10, 0:1]
    ya = ya + b3s
    yb = yb + b3s
    ya_ref[0] = ya
    yb_ref[0] = yb
    st_ref[0, 0, 0:1, 0:4] = _stats_row(
        [_tsum(ya), _tsum(ya * ya), _tsum(yb), _tsum(yb * yb)])


def _attn_body(xd_ref, ya_ref, yb_ref, pk_ref, out_ref):
    pk = pk_ref[...]
    ya = ya_ref[0]
    yb = yb_ref[0]
    sa = jnp.maximum(pk[10:11, 0:1] * ya + pk[10:11, 1:2], 0.0)
    na = jnp.maximum(pk[10:11, 2:3] * yb + pk[10:11, 3:4], 0.0)
    lg = sa + na
    lr = jnp.where(lg >= 0, lg, 0.01 * lg)
    mx = jnp.max(lr, axis=1, keepdims=True)
    e = jnp.exp(lr - mx)
    pr = e / jnp.sum(e, axis=1, keepdims=True)
    xd = [xd_ref[0, 0], xd_ref[0, 1], xd_ref[0, 2]]
    cols = []
    for o in range(16):
        ef = (pk[7:8, o:o + 1]
              + pk[4:5, o:o + 1] * xd[0]
              + pk[5:6, o:o + 1] * xd[1]
              + pk[6:7, o:o + 1] * xd[2])
        ef = jnp.maximum(ef, 0.0)
        cols.append(jnp.sum(pr * ef, axis=1, keepdims=True))
    v = jnp.concatenate(cols, axis=1)
    out_ref[0] = jnp.where(v > 0, v, jnp.exp(v) - 1.0)


def kernel(x, n_neighbor, W1, g1, be1, W2, b2, g2, be2, W3, b3, g3, be3):
    x = x.astype(jnp.float32)
    xt = jnp.transpose(x, (0, 2, 1))

    idx = pl.pallas_call(
        _knn_body,
        grid=(B, NT1),
        in_specs=[
            pl.BlockSpec((1, C, N), lambda b, t: (b, 0, 0)),
            pl.BlockSpec((1, T1, C), lambda b, t: (b, t, 0)),
        ],
        out_specs=pl.BlockSpec((1, T1, K), lambda b, t: (b, t, 0)),
        out_shape=jax.ShapeDtypeStruct((B, N, K), jnp.int32),
    )(x, xt)

    cidx = jnp.repeat(jnp.arange(N, dtype=jnp.int32), K)
    xdf, mom = _sc_gather(x, idx, cidx)
    xd = xdf.reshape(B, C, N, K)

    xd_spec = pl.BlockSpec((1, C, T, K), lambda b, t: (b, 0, t, 0))
    st_spec = pl.BlockSpec((1, 1, 8, 128), lambda b, t: (b, t, 0, 0))
    st_shape = jax.ShapeDtypeStruct((B, NT, 8, 128), jnp.float32)
    pk_spec = pl.BlockSpec((16, 16), lambda b, t: (0, 0))

    cnt = jnp.float32(B * N * K)
    s = jnp.sum(mom.reshape(32, 9, 16), axis=(0, 2))
    mu = s[:3] / cnt
    q = s[3:9] / cnt
    S = jnp.stack([
        jnp.stack([q[0], q[1], q[2]]),
        jnp.stack([q[1], q[3], q[4]]),
        jnp.stack([q[2], q[4], q[5]]),
    ])
    mean1 = W1 @ mu
    var1 = jnp.sum((W1 @ S) * W1, axis=1) - mean1 ** 2
    a1 = g1 / jnp.sqrt(var1 + 1e-5)
    A1 = a1[:, None] * W1
    beta1 = be1 - mean1 * a1
    z2 = W2 @ mu
    mean2 = z2 + b2
    var2 = jnp.sum((W2 @ S) * W2, axis=1) - z2 ** 2
    a2 = g2 / jnp.sqrt(var2 + 1e-5)
    A2 = a2[:, None] * W2
    beta2 = a2 * b2 + be2 - mean2 * a2

    pk = jnp.zeros((16, 16), jnp.float32)
    pk = pk.at[0:3, :].set(A1.T)
    pk = pk.at[3, :].set(beta1)
    pk = pk.at[4:7, :].set(A2.T)
    pk = pk.at[7, :].set(beta2)
    pk = pk.at[8, :].set(W3[0])
    pk = pk.at[9, 0].set(b3[0])

    ef, ya, yb, st2 = pl.pallas_call(
        _feat_body,
        grid=(B, NT),
        in_specs=[xd_spec, pk_spec],
        out_specs=[
            pl.BlockSpec((1, 16, T, K), lambda b, t: (b, 0, t, 0)),
            pl.BlockSpec((1, T, K), lambda b, t: (b, t, 0)),
            pl.BlockSpec((1, T, K), lambda b, t: (b, t, 0)),
            st_spec,
        ],
        out_shape=[
            jax.ShapeDtypeStruct((B, 16, N, K), jnp.float32),
            jax.ShapeDtypeStruct((B, N, K), jnp.float32),
            jax.ShapeDtypeStruct((B, N, K), jnp.float32),
            st_shape,
        ],
    )(xd, pk)

    s2 = jnp.sum(st2[:, :, 0, :4], axis=(0, 1))
    m3a = s2[0] / cnt
    v3a = s2[1] / cnt - m3a ** 2
    a3a = g3[0] / jnp.sqrt(v3a + 1e-5)
    b3a = be3[0] - m3a * a3a
    m3b = s2[2] / cnt
    v3b = s2[3] / cnt - m3b ** 2
    a3b = g3[0] / jnp.sqrt(v3b + 1e-5)
    b3b = be3[0] - m3b * a3b
    pk = pk.at[10, 0:4].set(jnp.stack([a3a, b3a, a3b, b3b]))

    out = pl.pallas_call(
        _attn_body,
        grid=(B, NT),
        in_specs=[
            xd_spec,
            pl.BlockSpec((1, T, K), lambda b, t: (b, t, 0)),
            pl.BlockSpec((1, T, K), lambda b, t: (b, t, 0)),
            pk_spec,
        ],
        out_specs=pl.BlockSpec((1, T, 16), lambda b, t: (b, t, 0)),
        out_shape=jax.ShapeDtypeStruct((B, N, 16), jnp.float32),
    )(xd, ya, yb, pk)

    ret = out[:, :, None, :]
    return (ret, ef)


# speedup vs baseline: 1.1416x; 1.0535x over previous
import functools

import jax
import jax.numpy as jnp
from jax import lax
from jax.experimental import pallas as pl
from jax.experimental.pallas import tpu as pltpu
from jax.experimental.pallas import tpu_sc as plsc

B, C, N, K = 16, 3, 2048, 20
T = 512
NT = N // T
T1 = 512
NT1 = N // T1
HALF = (N // 2) * K
_NEG = float("-inf")


def _tsum(x):
    return jnp.sum(jnp.sum(x, axis=1, keepdims=True), axis=0, keepdims=True)


def _stats_row(svals):
    return jnp.concatenate(svals, axis=1)


def _knn_body(x_ref, xtt_ref, idx_ref):
    xall = x_ref[0]
    xrow_t = xtt_ref[0]
    xxall = jnp.sum(xall * xall, axis=0, keepdims=True)
    xxrow = jnp.sum(xrow_t * xrow_t, axis=1, keepdims=True)
    inner = jax.lax.dot_general(
        xrow_t, xall, (((1,), (0,)), ((), ())),
        preferred_element_type=jnp.float32)
    inner = -2.0 * inner
    vals = (-xxall - inner) - xxrow

    lane_iota = jax.lax.broadcasted_iota(jnp.int32, (T1, N), 1)
    k_iota = jax.lax.broadcasted_iota(jnp.int32, (T1, K), 1)
    iacc = jnp.zeros((T1, K), jnp.int32)
    for j in range(K):
        a = jnp.argmax(vals, axis=1, keepdims=True).astype(jnp.int32)
        iacc = iacc + jnp.where(k_iota == j, a, 0)
        vals = jnp.where(lane_iota == a, _NEG, vals)
    idx_ref[0] = iacc


def _gather_tec(x_hbm, idx_hbm, cidx_hbm, xd_hbm, mom_hbm,
                xp0, xp1, xp2, idx_v, cid_v, ob0, ob1, ob2, mom_v):
    wid = lax.axis_index("s") * 2 + lax.axis_index("c")
    b = wid // 2
    base = (wid % 2) * HALF
    pltpu.sync_copy(x_hbm.at[pl.ds((b * C + 0) * N, N)], xp0)
    pltpu.sync_copy(x_hbm.at[pl.ds((b * C + 1) * N, N)], xp1)
    pltpu.sync_copy(x_hbm.at[pl.ds((b * C + 2) * N, N)], xp2)
    pltpu.sync_copy(idx_hbm.at[pl.ds(b * (N * K) + base, HALF)], idx_v)
    pltpu.sync_copy(cidx_hbm.at[pl.ds(base, HALF)], cid_v)

    def body(i, acc):
        sl = pl.ds(i * 16, 16)
        iv = idx_v[sl]
        civ = cid_v[sl]
        xd0 = plsc.load_gather(xp0, [civ]) - plsc.load_gather(xp0, [iv])
        xd1 = plsc.load_gather(xp1, [civ]) - plsc.load_gather(xp1, [iv])
        xd2 = plsc.load_gather(xp2, [civ]) - plsc.load_gather(xp2, [iv])
        ob0[sl] = xd0
        ob1[sl] = xd1
        ob2[sl] = xd2
        return (acc[0] + xd0, acc[1] + xd1, acc[2] + xd2,
                acc[3] + xd0 * xd0, acc[4] + xd0 * xd1, acc[5] + xd0 * xd2,
                acc[6] + xd1 * xd1, acc[7] + xd1 * xd2, acc[8] + xd2 * xd2)

    zero = jnp.zeros((16,), jnp.float32)
    acc = lax.fori_loop(0, HALF // 16, body, (zero,) * 9)
    for i in range(9):
        mom_v[pl.ds(i * 16, 16)] = acc[i]
    pltpu.sync_copy(ob0, xd_hbm.at[pl.ds((b * C + 0) * (N * K) + base, HALF)])
    pltpu.sync_copy(ob1, xd_hbm.at[pl.ds((b * C + 1) * (N * K) + base, HALF)])
    pltpu.sync_copy(ob2, xd_hbm.at[pl.ds((b * C + 2) * (N * K) + base, HALF)])
    pltpu.sync_copy(mom_v, mom_hbm.at[pl.ds(wid * 144, 144)])


def _sc_gather(x, idxf, cidx):
    f = pl.kernel(
        _gather_tec,
        out_type=[
            jax.ShapeDtypeStruct((B * C * N * K,), jnp.float32),
            jax.ShapeDtypeStruct((32 * 144,), jnp.float32),
        ],
        mesh=plsc.VectorSubcoreMesh(core_axis_name="c", subcore_axis_name="s"),
        compiler_params=pltpu.CompilerParams(needs_layout_passes=False),
        scratch_types=[
            pltpu.VMEM((N,), jnp.float32),
            pltpu.VMEM((N,), jnp.float32),
            pltpu.VMEM((N,), jnp.float32),
            pltpu.VMEM((HALF,), jnp.int32),
            pltpu.VMEM((HALF,), jnp.int32),
            pltpu.VMEM((HALF,), jnp.float32),
            pltpu.VMEM((HALF,), jnp.float32),
            pltpu.VMEM((HALF,), jnp.float32),
            pltpu.VMEM((144,), jnp.float32),
        ],
    )
    return f(x.reshape(B * C * N), idxf.reshape(B * N * K), cidx)


def _feat_body(xd_ref, pk_ref, ef_ref, ya_ref, yb_ref, st_ref):
    pk = pk_ref[...]
    xd = [xd_ref[0, 0], xd_ref[0, 1], xd_ref[0, 2]]
    ya = jnp.zeros((T, K), jnp.float32)
    yb = jnp.zeros((T, K), jnp.float32)
    for o in range(16):
        nf = (pk[3:4, o:o + 1]
              + pk[0:1, o:o + 1] * xd[0]
              + pk[1:2, o:o + 1] * xd[1]
              + pk[2:3, o:o + 1] * xd[2])
        ef = (pk[7:8, o:o + 1]
              + pk[4:5, o:o + 1] * xd[0]
              + pk[5:6, o:o + 1] * xd[1]
              + pk[6:7, o:o + 1] * xd[2])
        nf = jnp.maximum(nf, 0.0)
        ef = jnp.maximum(ef, 0.0)
        ef_ref[0, o] = ef
        w3o = pk[8:9, o:o + 1]
        ya = ya + w3o * nf
        yb = yb + w3o * ef
    b3s = pk[9:10, 0:1]
    ya = ya + b3s
    yb = yb + b3s
    ya_ref[0] = ya
    yb_ref[0] = yb
    st_ref[0, 0, 0:1, 0:4] = _stats_row(
        [_tsum(ya), _tsum(ya * ya), _tsum(yb), _tsum(yb * yb)])


def _attn_body(xd_ref, ya_ref, yb_ref, pk_ref, out_ref):
    pk = pk_ref[...]
    ya = ya_ref[0]
    yb = yb_ref[0]
    sa = jnp.maximum(pk[10:11, 0:1] * ya + pk[10:11, 1:2], 0.0)
    na = jnp.maximum(pk[10:11, 2:3] * yb + pk[10:11, 3:4], 0.0)
    lg = sa + na
    lr = jnp.where(lg >= 0, lg, 0.01 * lg)
    mx = jnp.max(lr, axis=1, keepdims=True)
    e = jnp.exp(lr - mx)
    pr = e / jnp.sum(e, axis=1, keepdims=True)
    xd = [xd_ref[0, 0], xd_ref[0, 1], xd_ref[0, 2]]
    cols = []
    for o in range(16):
        ef = (pk[7:8, o:o + 1]
              + pk[4:5, o:o + 1] * xd[0]
              + pk[5:6, o:o + 1] * xd[1]
              + pk[6:7, o:o + 1] * xd[2])
        ef = jnp.maximum(ef, 0.0)
        cols.append(jnp.sum(pr * ef, axis=1, keepdims=True))
    v = jnp.concatenate(cols, axis=1)
    out_ref[0] = jnp.where(v > 0, v, jnp.exp(v) - 1.0)


def kernel(x, n_neighbor, W1, g1, be1, W2, b2, g2, be2, W3, b3, g3, be3):
    x = x.astype(jnp.float32)
    xt = jnp.transpose(x, (0, 2, 1))

    idx = pl.pallas_call(
        _knn_body,
        grid=(B, NT1),
        in_specs=[
            pl.BlockSpec((1, C, N), lambda b, t: (b, 0, 0)),
            pl.BlockSpec((1, T1, C), lambda b, t: (b, t, 0)),
        ],
        out_specs=pl.BlockSpec((1, T1, K), lambda b, t: (b, t, 0)),
        out_shape=jax.ShapeDtypeStruct((B, N, K), jnp.int32),
    )(x, xt)

    cidx = jnp.repeat(jnp.arange(N, dtype=jnp.int32), K)
    xdf, mom = _sc_gather(x, idx, cidx)
    xd = xdf.reshape(B, C, N, K)

    xd_spec = pl.BlockSpec((1, C, T, K), lambda b, t: (b, 0, t, 0))
    st_spec = pl.BlockSpec((1, 1, 8, 128), lambda b, t: (b, t, 0, 0))
    st_shape = jax.ShapeDtypeStruct((B, NT, 8, 128), jnp.float32)
    pk_spec = pl.BlockSpec((16, 16), lambda b, t: (0, 0))

    cnt = jnp.float32(B * N * K)
    s = jnp.sum(mom.reshape(32, 9, 16), axis=(0, 2))
    mu = s[:3] / cnt
    q = s[3:9] / cnt
    S = jnp.stack([
        jnp.stack([q[0], q[1], q[2]]),
        jnp.stack([q[1], q[3], q[4]]),
        jnp.stack([q[2], q[4], q[5]]),
    ])
    mean1 = W1 @ mu
    var1 = jnp.sum((W1 @ S) * W1, axis=1) - mean1 ** 2
    a1 = g1 / jnp.sqrt(var1 + 1e-5)
    A1 = a1[:, None] * W1
    beta1 = be1 - mean1 * a1
    z2 = W2 @ mu
    mean2 = z2 + b2
    var2 = jnp.sum((W2 @ S) * W2, axis=1) - z2 ** 2
    a2 = g2 / jnp.sqrt(var2 + 1e-5)
    A2 = a2[:, None] * W2
    beta2 = a2 * b2 + be2 - mean2 * a2

    pk = jnp.zeros((16, 16), jnp.float32)
    pk = pk.at[0:3, :].set(A1.T)
    pk = pk.at[3, :].set(beta1)
    pk = pk.at[4:7, :].set(A2.T)
    pk = pk.at[7, :].set(beta2)
    pk = pk.at[8, :].set(W3[0])
    pk = pk.at[9, 0].set(b3[0])

    ef, ya, yb, st2 = pl.pallas_call(
        _feat_body,
        grid=(B, NT),
        in_specs=[xd_spec, pk_spec],
        out_specs=[
            pl.BlockSpec((1, 16, T, K), lambda b, t: (b, 0, t, 0)),
            pl.BlockSpec((1, T, K), lambda b, t: (b, t, 0)),
            pl.BlockSpec((1, T, K), lambda b, t: (b, t, 0)),
            st_spec,
        ],
        out_shape=[
            jax.ShapeDtypeStruct((B, 16, N, K), jnp.float32),
            jax.ShapeDtypeStruct((B, N, K), jnp.float32),
            jax.ShapeDtypeStruct((B, N, K), jnp.float32),
            st_shape,
        ],
    )(xd, pk)

    s2 = jnp.sum(st2[:, :, 0, :4], axis=(0, 1))
    m3a = s2[0] / cnt
    v3a = s2[1] / cnt - m3a ** 2
    a3a = g3[0] / jnp.sqrt(v3a + 1e-5)
    b3a = be3[0] - m3a * a3a
    m3b = s2[2] / cnt
    v3b = s2[3] / cnt - m3b ** 2
    a3b = g3[0] / jnp.sqrt(v3b + 1e-5)
    b3b = be3[0] - m3b * a3b
    pk = pk.at[10, 0:4].set(jnp.stack([a3a, b3a, a3b, b3b]))

    out = pl.pallas_call(
        _attn_body,
        grid=(B, NT),
        in_specs=[
            xd_spec,
            pl.BlockSpec((1, T, K), lambda b, t: (b, t, 0)),
            pl.BlockSpec((1, T, K), lambda b, t: (b, t, 0)),
            pk_spec,
        ],
        out_specs=pl.BlockSpec((1, T, 16), lambda b, t: (b, t, 0)),
        out_shape=jax.ShapeDtypeStruct((B, N, 16), jnp.float32),
    )(xd, ya, yb, pk)

    ret = out[:, :, None, :]
    return (ret, ef)


# speedup vs baseline: 1.1671x; 1.0223x over previous
import functools

import jax
import jax.numpy as jnp
from jax import lax
from jax.experimental import pallas as pl
from jax.experimental.pallas import tpu as pltpu
from jax.experimental.pallas import tpu_sc as plsc

B, C, N, K = 16, 3, 2048, 20
T = 1024
NT = N // T
T1 = 512
NT1 = N // T1
HALF = (N // 2) * K
_NEG = float("-inf")


def _tsum(x):
    return jnp.sum(jnp.sum(x, axis=1, keepdims=True), axis=0, keepdims=True)


def _stats_row(svals):
    return jnp.concatenate(svals, axis=1)


def _knn_body(x_ref, xtt_ref, idx_ref):
    xall = x_ref[0]
    xrow_t = xtt_ref[0]
    xxall = jnp.sum(xall * xall, axis=0, keepdims=True)
    xxrow = jnp.sum(xrow_t * xrow_t, axis=1, keepdims=True)
    inner = jax.lax.dot_general(
        xrow_t, xall, (((1,), (0,)), ((), ())),
        preferred_element_type=jnp.float32)
    inner = -2.0 * inner
    vals = (-xxall - inner) - xxrow

    lane_iota = jax.lax.broadcasted_iota(jnp.int32, (T1, N), 1)
    k_iota = jax.lax.broadcasted_iota(jnp.int32, (T1, K), 1)
    iacc = jnp.zeros((T1, K), jnp.int32)
    for j in range(K):
        a = jnp.argmax(vals, axis=1, keepdims=True).astype(jnp.int32)
        iacc = iacc + jnp.where(k_iota == j, a, 0)
        vals = jnp.where(lane_iota == a, _NEG, vals)
    idx_ref[0] = iacc


def _gather_tec(x_hbm, idx_hbm, cidx_hbm, xd_hbm, mom_hbm,
                xp0, xp1, xp2, idx_v, cid_v, ob0, ob1, ob2, mom_v):
    wid = lax.axis_index("s") * 2 + lax.axis_index("c")
    b = wid // 2
    base = (wid % 2) * HALF
    pltpu.sync_copy(x_hbm.at[pl.ds((b * C + 0) * N, N)], xp0)
    pltpu.sync_copy(x_hbm.at[pl.ds((b * C + 1) * N, N)], xp1)
    pltpu.sync_copy(x_hbm.at[pl.ds((b * C + 2) * N, N)], xp2)
    pltpu.sync_copy(idx_hbm.at[pl.ds(b * (N * K) + base, HALF)], idx_v)
    pltpu.sync_copy(cidx_hbm.at[pl.ds(base, HALF)], cid_v)

    def body(i, acc):
        sl = pl.ds(i * 16, 16)
        iv = idx_v[sl]
        civ = cid_v[sl]
        xd0 = plsc.load_gather(xp0, [civ]) - plsc.load_gather(xp0, [iv])
        xd1 = plsc.load_gather(xp1, [civ]) - plsc.load_gather(xp1, [iv])
        xd2 = plsc.load_gather(xp2, [civ]) - plsc.load_gather(xp2, [iv])
        ob0[sl] = xd0
        ob1[sl] = xd1
        ob2[sl] = xd2
        return (acc[0] + xd0, acc[1] + xd1, acc[2] + xd2,
                acc[3] + xd0 * xd0, acc[4] + xd0 * xd1, acc[5] + xd0 * xd2,
                acc[6] + xd1 * xd1, acc[7] + xd1 * xd2, acc[8] + xd2 * xd2)

    zero = jnp.zeros((16,), jnp.float32)
    acc = lax.fori_loop(0, HALF // 16, body, (zero,) * 9)
    for i in range(9):
        mom_v[pl.ds(i * 16, 16)] = acc[i]
    pltpu.sync_copy(ob0, xd_hbm.at[pl.ds((b * C + 0) * (N * K) + base, HALF)])
    pltpu.sync_copy(ob1, xd_hbm.at[pl.ds((b * C + 1) * (N * K) + base, HALF)])
    pltpu.sync_copy(ob2, xd_hbm.at[pl.ds((b * C + 2) * (N * K) + base, HALF)])
    pltpu.sync_copy(mom_v, mom_hbm.at[pl.ds(wid * 144, 144)])


def _sc_gather(x, idxf, cidx):
    f = pl.kernel(
        _gather_tec,
        out_type=[
            jax.ShapeDtypeStruct((B * C * N * K,), jnp.float32),
            jax.ShapeDtypeStruct((32 * 144,), jnp.float32),
        ],
        mesh=plsc.VectorSubcoreMesh(core_axis_name="c", subcore_axis_name="s"),
        compiler_params=pltpu.CompilerParams(needs_layout_passes=False),
        scratch_types=[
            pltpu.VMEM((N,), jnp.float32),
            pltpu.VMEM((N,), jnp.float32),
            pltpu.VMEM((N,), jnp.float32),
            pltpu.VMEM((HALF,), jnp.int32),
            pltpu.VMEM((HALF,), jnp.int32),
            pltpu.VMEM((HALF,), jnp.float32),
            pltpu.VMEM((HALF,), jnp.float32),
            pltpu.VMEM((HALF,), jnp.float32),
            pltpu.VMEM((144,), jnp.float32),
        ],
    )
    return f(x.reshape(B * C * N), idxf.reshape(B * N * K), cidx)


def _feat_body(xd_ref, pk_ref, ef_ref, ya_ref, yb_ref, st_ref):
    pk = pk_ref[...]
    xd = [xd_ref[0, 0], xd_ref[0, 1], xd_ref[0, 2]]
    ya = jnp.zeros((T, K), jnp.float32)
    yb = jnp.zeros((T, K), jnp.float32)
    for o in range(16):
        nf = (pk[3:4, o:o + 1]
              + pk[0:1, o:o + 1] * xd[0]
              + pk[1:2, o:o + 1] * xd[1]
              + pk[2:3, o:o + 1] * xd[2])
        ef = (pk[7:8, o:o + 1]
              + pk[4:5, o:o + 1] * xd[0]
              + pk[5:6, o:o + 1] * xd[1]
              + pk[6:7, o:o + 1] * xd[2])
        nf = jnp.maximum(nf, 0.0)
        ef = jnp.maximum(ef, 0.0)
        ef_ref[0, o] = ef
        w3o = pk[8:9, o:o + 1]
        ya = ya + w3o * nf
        yb = yb + w3o * ef
    b3s = pk[9:10, 0:1]
    ya = ya + b3s
    yb = yb + b3s
    ya_ref[0] = ya
    yb_ref[0] = yb
    st_ref[0, 0, 0:1, 0:4] = _stats_row(
        [_tsum(ya), _tsum(ya * ya), _tsum(yb), _tsum(yb * yb)])


def _attn_body(xd_ref, ya_ref, yb_ref, pk_ref, out_ref):
    pk = pk_ref[...]
    ya = ya_ref[0]
    yb = yb_ref[0]
    sa = jnp.maximum(pk[10:11, 0:1] * ya + pk[10:11, 1:2], 0.0)
    na = jnp.maximum(pk[10:11, 2:3] * yb + pk[10:11, 3:4], 0.0)
    lg = sa + na
    lr = jnp.where(lg >= 0, lg, 0.01 * lg)
    mx = jnp.max(lr, axis=1, keepdims=True)
    e = jnp.exp(lr - mx)
    pr = e / jnp.sum(e, axis=1, keepdims=True)
    xd = [xd_ref[0, 0], xd_ref[0, 1], xd_ref[0, 2]]
    cols = []
    for o in range(16):
        ef = (pk[7:8, o:o + 1]
              + pk[4:5, o:o + 1] * xd[0]
              + pk[5:6, o:o + 1] * xd[1]
              + pk[6:7, o:o + 1] * xd[2])
        ef = jnp.maximum(ef, 0.0)
        cols.append(jnp.sum(pr * ef, axis=1, keepdims=True))
    v = jnp.concatenate(cols, axis=1)
    out_ref[0] = jnp.where(v > 0, v, jnp.exp(v) - 1.0)


def kernel(x, n_neighbor, W1, g1, be1, W2, b2, g2, be2, W3, b3, g3, be3):
    x = x.astype(jnp.float32)
    xt = jnp.transpose(x, (0, 2, 1))

    idx = pl.pallas_call(
        _knn_body,
        grid=(B, NT1),
        in_specs=[
            pl.BlockSpec((1, C, N), lambda b, t: (b, 0, 0)),
            pl.BlockSpec((1, T1, C), lambda b, t: (b, t, 0)),
        ],
        out_specs=pl.BlockSpec((1, T1, K), lambda b, t: (b, t, 0)),
        out_shape=jax.ShapeDtypeStruct((B, N, K), jnp.int32),
    )(x, xt)

    cidx = jnp.repeat(jnp.arange(N, dtype=jnp.int32), K)
    xdf, mom = _sc_gather(x, idx, cidx)
    xd = xdf.reshape(B, C, N, K)

    xd_spec = pl.BlockSpec((1, C, T, K), lambda b, t: (b, 0, t, 0))
    st_spec = pl.BlockSpec((1, 1, 8, 128), lambda b, t: (b, t, 0, 0))
    st_shape = jax.ShapeDtypeStruct((B, NT, 8, 128), jnp.float32)
    pk_spec = pl.BlockSpec((16, 16), lambda b, t: (0, 0))

    cnt = jnp.float32(B * N * K)
    s = jnp.sum(mom.reshape(32, 9, 16), axis=(0, 2))
    mu = s[:3] / cnt
    q = s[3:9] / cnt
    S = jnp.stack([
        jnp.stack([q[0], q[1], q[2]]),
        jnp.stack([q[1], q[3], q[4]]),
        jnp.stack([q[2], q[4], q[5]]),
    ])
    mean1 = W1 @ mu
    var1 = jnp.sum((W1 @ S) * W1, axis=1) - mean1 ** 2
    a1 = g1 / jnp.sqrt(var1 + 1e-5)
    A1 = a1[:, None] * W1
    beta1 = be1 - mean1 * a1
    z2 = W2 @ mu
    mean2 = z2 + b2
    var2 = jnp.sum((W2 @ S) * W2, axis=1) - z2 ** 2
    a2 = g2 / jnp.sqrt(var2 + 1e-5)
    A2 = a2[:, None] * W2
    beta2 = a2 * b2 + be2 - mean2 * a2

    pk = jnp.zeros((16, 16), jnp.float32)
    pk = pk.at[0:3, :].set(A1.T)
    pk = pk.at[3, :].set(beta1)
    pk = pk.at[4:7, :].set(A2.T)
    pk = pk.at[7, :].set(beta2)
    pk = pk.at[8, :].set(W3[0])
    pk = pk.at[9, 0].set(b3[0])

    ef, ya, yb, st2 = pl.pallas_call(
        _feat_body,
        grid=(B, NT),
        in_specs=[xd_spec, pk_spec],
        out_specs=[
            pl.BlockSpec((1, 16, T, K), lambda b, t: (b, 0, t, 0)),
            pl.BlockSpec((1, T, K), lambda b, t: (b, t, 0)),
            pl.BlockSpec((1, T, K), lambda b, t: (b, t, 0)),
            st_spec,
        ],
        out_shape=[
            jax.ShapeDtypeStruct((B, 16, N, K), jnp.float32),
            jax.ShapeDtypeStruct((B, N, K), jnp.float32),
            jax.ShapeDtypeStruct((B, N, K), jnp.float32),
            st_shape,
        ],
    )(xd, pk)

    s2 = jnp.sum(st2[:, :, 0, :4], axis=(0, 1))
    m3a = s2[0] / cnt
    v3a = s2[1] / cnt - m3a ** 2
    a3a = g3[0] / jnp.sqrt(v3a + 1e-5)
    b3a = be3[0] - m3a * a3a
    m3b = s2[2] / cnt
    v3b = s2[3] / cnt - m3b ** 2
    a3b = g3[0] / jnp.sqrt(v3b + 1e-5)
    b3b = be3[0] - m3b * a3b
    pk = pk.at[10, 0:4].set(jnp.stack([a3a, b3a, a3b, b3b]))

    out = pl.pallas_call(
        _attn_body,
        grid=(B, NT),
        in_specs=[
            xd_spec,
            pl.BlockSpec((1, T, K), lambda b, t: (b, t, 0)),
            pl.BlockSpec((1, T, K), lambda b, t: (b, t, 0)),
            pk_spec,
        ],
        out_specs=pl.BlockSpec((1, T, 16), lambda b, t: (b, t, 0)),
        out_shape=jax.ShapeDtypeStruct((B, N, 16), jnp.float32),
    )(xd, ya, yb, pk)

    ret = out[:, :, None, :]
    return (ret, ef)


# speedup vs baseline: 1.1843x; 1.0148x over previous
import functools

import jax
import jax.numpy as jnp
from jax import lax
from jax.experimental import pallas as pl
from jax.experimental.pallas import tpu as pltpu
from jax.experimental.pallas import tpu_sc as plsc

B, C, N, K = 16, 3, 2048, 20
T = 2048
NT = N // T
T1 = 512
NT1 = N // T1
HALF = (N // 2) * K
_NEG = float("-inf")


def _tsum(x):
    return jnp.sum(jnp.sum(x, axis=1, keepdims=True), axis=0, keepdims=True)


def _stats_row(svals):
    return jnp.concatenate(svals, axis=1)


def _knn_body(x_ref, xtt_ref, idx_ref):
    xall = x_ref[0]
    xrow_t = xtt_ref[0]
    xxall = jnp.sum(xall * xall, axis=0, keepdims=True)
    xxrow = jnp.sum(xrow_t * xrow_t, axis=1, keepdims=True)
    inner = jax.lax.dot_general(
        xrow_t, xall, (((1,), (0,)), ((), ())),
        preferred_element_type=jnp.float32)
    inner = -2.0 * inner
    vals = (-xxall - inner) - xxrow

    lane_iota = jax.lax.broadcasted_iota(jnp.int32, (T1, N), 1)
    k_iota = jax.lax.broadcasted_iota(jnp.int32, (T1, K), 1)
    iacc = jnp.zeros((T1, K), jnp.int32)
    for j in range(K):
        a = jnp.argmax(vals, axis=1, keepdims=True).astype(jnp.int32)
        iacc = iacc + jnp.where(k_iota == j, a, 0)
        vals = jnp.where(lane_iota == a, _NEG, vals)
    idx_ref[0] = iacc


def _gather_tec(x_hbm, idx_hbm, cidx_hbm, xd_hbm, mom_hbm,
                xp0, xp1, xp2, idx_v, cid_v, ob0, ob1, ob2, mom_v):
    wid = lax.axis_index("s") * 2 + lax.axis_index("c")
    b = wid // 2
    base = (wid % 2) * HALF
    pltpu.sync_copy(x_hbm.at[pl.ds((b * C + 0) * N, N)], xp0)
    pltpu.sync_copy(x_hbm.at[pl.ds((b * C + 1) * N, N)], xp1)
    pltpu.sync_copy(x_hbm.at[pl.ds((b * C + 2) * N, N)], xp2)
    pltpu.sync_copy(idx_hbm.at[pl.ds(b * (N * K) + base, HALF)], idx_v)
    pltpu.sync_copy(cidx_hbm.at[pl.ds(base, HALF)], cid_v)

    def body(i, acc):
        sl = pl.ds(i * 16, 16)
        iv = idx_v[sl]
        civ = cid_v[sl]
        xd0 = plsc.load_gather(xp0, [civ]) - plsc.load_gather(xp0, [iv])
        xd1 = plsc.load_gather(xp1, [civ]) - plsc.load_gather(xp1, [iv])
        xd2 = plsc.load_gather(xp2, [civ]) - plsc.load_gather(xp2, [iv])
        ob0[sl] = xd0
        ob1[sl] = xd1
        ob2[sl] = xd2
        return (acc[0] + xd0, acc[1] + xd1, acc[2] + xd2,
                acc[3] + xd0 * xd0, acc[4] + xd0 * xd1, acc[5] + xd0 * xd2,
                acc[6] + xd1 * xd1, acc[7] + xd1 * xd2, acc[8] + xd2 * xd2)

    zero = jnp.zeros((16,), jnp.float32)
    acc = lax.fori_loop(0, HALF // 16, body, (zero,) * 9)
    for i in range(9):
        mom_v[pl.ds(i * 16, 16)] = acc[i]
    pltpu.sync_copy(ob0, xd_hbm.at[pl.ds((b * C + 0) * (N * K) + base, HALF)])
    pltpu.sync_copy(ob1, xd_hbm.at[pl.ds((b * C + 1) * (N * K) + base, HALF)])
    pltpu.sync_copy(ob2, xd_hbm.at[pl.ds((b * C + 2) * (N * K) + base, HALF)])
    pltpu.sync_copy(mom_v, mom_hbm.at[pl.ds(wid * 144, 144)])


def _sc_gather(x, idxf, cidx):
    f = pl.kernel(
        _gather_tec,
        out_type=[
            jax.ShapeDtypeStruct((B * C * N * K,), jnp.float32),
            jax.ShapeDtypeStruct((32 * 144,), jnp.float32),
        ],
        mesh=plsc.VectorSubcoreMesh(core_axis_name="c", subcore_axis_name="s"),
        compiler_params=pltpu.CompilerParams(needs_layout_passes=False),
        scratch_types=[
            pltpu.VMEM((N,), jnp.float32),
            pltpu.VMEM((N,), jnp.float32),
            pltpu.VMEM((N,), jnp.float32),
            pltpu.VMEM((HALF,), jnp.int32),
            pltpu.VMEM((HALF,), jnp.int32),
            pltpu.VMEM((HALF,), jnp.float32),
            pltpu.VMEM((HALF,), jnp.float32),
            pltpu.VMEM((HALF,), jnp.float32),
            pltpu.VMEM((144,), jnp.float32),
        ],
    )
    return f(x.reshape(B * C * N), idxf.reshape(B * N * K), cidx)


def _feat_body(xd_ref, pk_ref, ef_ref, ya_ref, yb_ref, st_ref):
    pk = pk_ref[...]
    xd = [xd_ref[0, 0], xd_ref[0, 1], xd_ref[0, 2]]
    ya = jnp.zeros((T, K), jnp.float32)
    yb = jnp.zeros((T, K), jnp.float32)
    for o in range(16):
        nf = (pk[3:4, o:o + 1]
              + pk[0:1, o:o + 1] * xd[0]
              + pk[1:2, o:o + 1] * xd[1]
              + pk[2:3, o:o + 1] * xd[2])
        ef = (pk[7:8, o:o + 1]
              + pk[4:5, o:o + 1] * xd[0]
              + pk[5:6, o:o + 1] * xd[1]
              + pk[6:7, o:o + 1] * xd[2])
        nf = jnp.maximum(nf, 0.0)
        ef = jnp.maximum(ef, 0.0)
        ef_ref[0, o] = ef
        w3o = pk[8:9, o:o + 1]
        ya = ya + w3o * nf
        yb = yb + w3o * ef
    b3s = pk[9:10, 0:1]
    ya = ya + b3s
    yb = yb + b3s
    ya_ref[0] = ya
    yb_ref[0] = yb
    st_ref[0, 0, 0:1, 0:4] = _stats_row(
        [_tsum(ya), _tsum(ya * ya), _tsum(yb), _tsum(yb * yb)])


def _attn_body(xd_ref, ya_ref, yb_ref, pk_ref, out_ref):
    pk = pk_ref[...]
    ya = ya_ref[0]
    yb = yb_ref[0]
    sa = jnp.maximum(pk[10:11, 0:1] * ya + pk[10:11, 1:2], 0.0)
    na = jnp.maximum(pk[10:11, 2:3] * yb + pk[10:11, 3:4], 0.0)
    lg = sa + na
    lr = jnp.where(lg >= 0, lg, 0.01 * lg)
    mx = jnp.max(lr, axis=1, keepdims=True)
    e = jnp.exp(lr - mx)
    pr = e / jnp.sum(e, axis=1, keepdims=True)
    xd = [xd_ref[0, 0], xd_ref[0, 1], xd_ref[0, 2]]
    cols = []
    for o in range(16):
        ef = (pk[7:8, o:o + 1]
              + pk[4:5, o:o + 1] * xd[0]
              + pk[5:6, o:o + 1] * xd[1]
              + pk[6:7, o:o + 1] * xd[2])
        ef = jnp.maximum(ef, 0.0)
        cols.append(jnp.sum(pr * ef, axis=1, keepdims=True))
    v = jnp.concatenate(cols, axis=1)
    out_ref[0] = jnp.where(v > 0, v, jnp.exp(v) - 1.0)


def kernel(x, n_neighbor, W1, g1, be1, W2, b2, g2, be2, W3, b3, g3, be3):
    x = x.astype(jnp.float32)
    xt = jnp.transpose(x, (0, 2, 1))

    idx = pl.pallas_call(
        _knn_body,
        grid=(B, NT1),
        in_specs=[
            pl.BlockSpec((1, C, N), lambda b, t: (b, 0, 0)),
            pl.BlockSpec((1, T1, C), lambda b, t: (b, t, 0)),
        ],
        out_specs=pl.BlockSpec((1, T1, K), lambda b, t: (b, t, 0)),
        out_shape=jax.ShapeDtypeStruct((B, N, K), jnp.int32),
    )(x, xt)

    cidx = jnp.repeat(jnp.arange(N, dtype=jnp.int32), K)
    xdf, mom = _sc_gather(x, idx, cidx)
    xd = xdf.reshape(B, C, N, K)

    xd_spec = pl.BlockSpec((1, C, T, K), lambda b, t: (b, 0, t, 0))
    st_spec = pl.BlockSpec((1, 1, 8, 128), lambda b, t: (b, t, 0, 0))
    st_shape = jax.ShapeDtypeStruct((B, NT, 8, 128), jnp.float32)
    pk_spec = pl.BlockSpec((16, 16), lambda b, t: (0, 0))

    cnt = jnp.float32(B * N * K)
    s = jnp.sum(mom.reshape(32, 9, 16), axis=(0, 2))
    mu = s[:3] / cnt
    q = s[3:9] / cnt
    S = jnp.stack([
        jnp.stack([q[0], q[1], q[2]]),
        jnp.stack([q[1], q[3], q[4]]),
        jnp.stack([q[2], q[4], q[5]]),
    ])
    mean1 = W1 @ mu
    var1 = jnp.sum((W1 @ S) * W1, axis=1) - mean1 ** 2
    a1 = g1 / jnp.sqrt(var1 + 1e-5)
    A1 = a1[:, None] * W1
    beta1 = be1 - mean1 * a1
    z2 = W2 @ mu
    mean2 = z2 + b2
    var2 = jnp.sum((W2 @ S) * W2, axis=1) - z2 ** 2
    a2 = g2 / jnp.sqrt(var2 + 1e-5)
    A2 = a2[:, None] * W2
    beta2 = a2 * b2 + be2 - mean2 * a2

    pk = jnp.zeros((16, 16), jnp.float32)
    pk = pk.at[0:3, :].set(A1.T)
    pk = pk.at[3, :].set(beta1)
    pk = pk.at[4:7, :].set(A2.T)
    pk = pk.at[7, :].set(beta2)
    pk = pk.at[8, :].set(W3[0])
    pk = pk.at[9, 0].set(b3[0])

    ef, ya, yb, st2 = pl.pallas_call(
        _feat_body,
        grid=(B, NT),
        in_specs=[xd_spec, pk_spec],
        out_specs=[
            pl.BlockSpec((1, 16, T, K), lambda b, t: (b, 0, t, 0)),
            pl.BlockSpec((1, T, K), lambda b, t: (b, t, 0)),
            pl.BlockSpec((1, T, K), lambda b, t: (b, t, 0)),
            st_spec,
        ],
        out_shape=[
            jax.ShapeDtypeStruct((B, 16, N, K), jnp.float32),
            jax.ShapeDtypeStruct((B, N, K), jnp.float32),
            jax.ShapeDtypeStruct((B, N, K), jnp.float32),
            st_shape,
        ],
    )(xd, pk)

    s2 = jnp.sum(st2[:, :, 0, :4], axis=(0, 1))
    m3a = s2[0] / cnt
    v3a = s2[1] / cnt - m3a ** 2
    a3a = g3[0] / jnp.sqrt(v3a + 1e-5)
    b3a = be3[0] - m3a * a3a
    m3b = s2[2] / cnt
    v3b = s2[3] / cnt - m3b ** 2
    a3b = g3[0] / jnp.sqrt(v3b + 1e-5)
    b3b = be3[0] - m3b * a3b
    pk = pk.at[10, 0:4].set(jnp.stack([a3a, b3a, a3b, b3b]))

    out = pl.pallas_call(
        _attn_body,
        grid=(B, NT),
        in_specs=[
            xd_spec,
            pl.BlockSpec((1, T, K), lambda b, t: (b, t, 0)),
            pl.BlockSpec((1, T, K), lambda b, t: (b, t, 0)),
            pk_spec,
        ],
        out_specs=pl.BlockSpec((1, T, 16), lambda b, t: (b, t, 0)),
        out_shape=jax.ShapeDtypeStruct((B, N, 16), jnp.float32),
    )(xd, ya, yb, pk)

    ret = out[:, :, None, :]
    return (ret, ef)
